# baseline (speedup 1.0000x reference)
import jax
import jax.numpy as jnp
from jax.experimental import pallas as pl
from jax.experimental.pallas import tpu as pltpu

_PW = 256
_NP = 4


def _vq_kernel(xt_ref, vt_ref, out_ref):
    nb = out_ref.shape[0]
    v = jnp.transpose(vt_ref[...])
    hb2 = 0.5 * jnp.sum(v * v, axis=-1, keepdims=True)

    for p in range(_NP):
        for b in range(nb):
            x = xt_ref[b, :, pl.ds(p * _PW, _PW)]
            ab = jax.lax.dot_general(
                v, x, (((1,), (0,)), ((), ())),
                preferred_element_type=jnp.float32)
            s = hb2 - ab
            out_ref[b, pl.ds(p * _PW, _PW)] = (
                jnp.argmin(s, axis=0).astype(jnp.int32))


def kernel(patches, vocab):
    b, n, dim = patches.shape
    k = vocab.shape[0]
    xt = jnp.transpose(patches, (0, 2, 1))
    vt = jnp.transpose(vocab)

    def call(xt_half):
        return pl.pallas_call(
            _vq_kernel,
            in_specs=[
                pl.BlockSpec(memory_space=pltpu.MemorySpace.VMEM),
                pl.BlockSpec(memory_space=pltpu.MemorySpace.VMEM),
            ],
            out_specs=pl.BlockSpec(memory_space=pltpu.MemorySpace.VMEM),
            out_shape=jax.ShapeDtypeStruct((xt_half.shape[0], n), jnp.int32),
        )(xt_half, vt)

    h = b // 2
    return jnp.concatenate([call(xt[:h]), call(xt[h:])], axis=0)


# speedup vs baseline: 1.6200x; 1.6200x over previous
import jax
import jax.numpy as jnp
from jax.experimental import pallas as pl
from jax.experimental.pallas import tpu as pltpu

_PW = 256
_NP = 4


def _vq_kernel(xt_ref, vt_ref, out_ref):
    nb = out_ref.shape[0]
    v = jnp.transpose(vt_ref[...])
    hb2 = 0.5 * jnp.sum(v * v, axis=-1, keepdims=True)

    for p in range(_NP):
        for b in range(nb):
            x = xt_ref[b, :, pl.ds(p * _PW, _PW)]
            ab = jax.lax.dot_general(
                v, x, (((1,), (0,)), ((), ())),
                preferred_element_type=jnp.float32)
            s = hb2 - ab
            out_ref[b, pl.ds(p * _PW, _PW)] = (
                jnp.argmin(s, axis=0).astype(jnp.int32))


def kernel(patches, vocab):
    b, n, dim = patches.shape
    k = vocab.shape[0]
    xt = jnp.transpose(patches, (0, 2, 1))
    vt = jnp.transpose(vocab)

    out = pl.pallas_call(
        _vq_kernel,
        in_specs=[
            pl.BlockSpec(memory_space=pltpu.MemorySpace.VMEM),
            pl.BlockSpec(memory_space=pltpu.MemorySpace.VMEM),
        ],
        out_specs=pl.BlockSpec(memory_space=pltpu.MemorySpace.VMEM),
        out_shape=jax.ShapeDtypeStruct((b, n), jnp.int32),
    )(xt, vt)
    return out
